# baseline (speedup 1.0000x reference)
import jax
import jax.numpy as jnp
from jax import lax
from jax.experimental import pallas as pl
from jax.experimental.pallas import tpu as pltpu
from jax.experimental.pallas import tpu_sc as plsc

ROWS = 1_000_000
D = 64
NIDX = 16384

NC = 2
NS = 16
NW = NC * NS
RUNS_PER_TILE = NIDX // NW
SLAB = 128
NSLAB = RUNS_PER_TILE // SLAB
NSLABS_TOT = NIDX // SLAB
PREC = 16
NCAND = 2 * NIDX + 16

COPY_CHUNK = 125_000
NCHUNK = ROWS // COPY_CHUNK


def _lane(vec, j):
    return lax.squeeze(lax.slice(vec, [j], [j + 1]), [0])


def _copy_body(a_hbm, o_hbm, *sems):
    for c in range(NCHUNK):
        pltpu.async_copy(
            a_hbm.at[pl.ds(c * COPY_CHUNK, COPY_CHUNK)],
            o_hbm.at[pl.ds(c * COPY_CHUNK, COPY_CHUNK)],
            sems[c])
    for c in range(NCHUNK):
        pltpu.make_async_copy(
            a_hbm.at[pl.ds(0, COPY_CHUNK)],
            o_hbm.at[pl.ds(0, COPY_CHUNK)],
            sems[c]).wait()


def _tc_copy(A):
    return pl.pallas_call(
        _copy_body,
        in_specs=[pl.BlockSpec(memory_space=pl.ANY)],
        out_specs=pl.BlockSpec(memory_space=pl.ANY),
        out_shape=jax.ShapeDtypeStruct((ROWS, D), jnp.float32),
        scratch_shapes=[pltpu.SemaphoreType.DMA] * NCHUNK,
    )(A)


def _sc_body(order_hbm, rid_hbm, rr_hbm, rec_hbm, b_hbm, out_hbm,
             odbuf, ridbuf, recbuf, rbuf, tbuf, bbuf, gbuf, semB):
    wid = lax.axis_index("s") * NC + lax.axis_index("c")

    pltpu.sync_copy(order_hbm, odbuf)
    pltpu.sync_copy(rid_hbm, ridbuf)
    pltpu.sync_copy(rec_hbm.at[pl.ds(wid * NSLAB * PREC, NSLAB * PREC)],
                    recbuf)
    zeros16 = jnp.zeros((16,), jnp.float32)

    def slab_body(sl, carry):
        rec = recbuf[pl.ds(sl * PREC, 16)]
        ps = _lane(rec, 0)
        pe = _lane(rec, 1)
        rid0 = wid * RUNS_PER_TILE + sl * SLAB

        def zero_body(r, zc):
            for cg in range(4):
                tbuf[r, pl.ds(cg * 16, 16)] = zeros16
            return zc
        lax.fori_loop(0, SLAB, zero_body, 0)

        pltpu.sync_copy(rr_hbm.at[pl.ds(rid0, SLAB)], rbuf)

        def batch_body(b, bc):
            bb = b * SLAB
            pltpu.async_copy(
                b_hbm.at[odbuf.at[pl.ds(bb, SLAB)]], bbuf, semB).wait()
            for sub in range(SLAB // 16):
                rv = ridbuf[pl.ds(bb + sub * 16, 16)]
                for j in range(16):
                    pos = bb + sub * 16 + j
                    cond = jnp.logical_and(pos >= ps, pos < pe)

                    @pl.when(cond)
                    def _(sub=sub, j=j, rv=rv):
                        lr = _lane(rv, j) - rid0
                        for cg in range(4):
                            x = bbuf[sub * 16 + j, pl.ds(cg * 16, 16)]
                            plsc.addupdate(
                                tbuf.at[lr, pl.ds(cg * 16, 16)], x)
            return bc

        lax.fori_loop(ps // SLAB, (pe + SLAB - 1) // SLAB, batch_body, 0)

        pltpu.async_copy(out_hbm.at[rbuf], gbuf, semB).wait()

        def add_body(r, ac):
            for cg in range(4):
                x = gbuf[r, pl.ds(cg * 16, 16)]
                plsc.addupdate(tbuf.at[r, pl.ds(cg * 16, 16)], x)
            return ac
        lax.fori_loop(0, SLAB, add_body, 0)

        pltpu.async_copy(tbuf, out_hbm.at[rbuf], semB).wait()
        return carry

    lax.fori_loop(0, NSLAB, slab_body, 0)


def _sc_scatter(order, rid, run_rows, recs, B, out_ref):
    mesh = plsc.VectorSubcoreMesh(
        core_axis_name="c", subcore_axis_name="s",
        num_cores=NC, num_subcores=NS)
    f = pl.kernel(
        _sc_body,
        out_type=(),
        mesh=mesh,
        compiler_params=pltpu.CompilerParams(use_tc_tiling_on_sc=False),
        scratch_types=[
            pltpu.VMEM((NIDX,), jnp.int32),
            pltpu.VMEM((NIDX,), jnp.int32),
            pltpu.VMEM((NSLAB * PREC,), jnp.int32),
            pltpu.VMEM((SLAB,), jnp.int32),
            pltpu.VMEM((SLAB, D), jnp.float32),
            pltpu.VMEM((SLAB, D), jnp.float32),
            pltpu.VMEM((SLAB, D), jnp.float32),
            pltpu.SemaphoreType.DMA,
        ],
    )
    f(order, rid, run_rows, recs, B, out_ref)


@jax.jit
def _scatter_add(index, A, B):
    iota = jnp.arange(NIDX, dtype=jnp.int32)
    sidx, order = lax.sort([index, iota], num_keys=1)
    is_start = jnp.concatenate(
        [jnp.ones((1,), jnp.bool_), sidx[1:] != sidx[:-1]])
    rid = jnp.cumsum(is_start.astype(jnp.int32)) - 1
    nruns = rid[NIDX - 1] + 1
    keys = jnp.where(is_start, rid, NIDX)
    run_rows = lax.sort([keys, sidx], num_keys=1)[1]
    cand = jnp.arange(NCAND, dtype=jnp.int32)
    kall = jnp.concatenate([sidx, cand])
    tag = jnp.concatenate(
        [jnp.zeros((NIDX,), jnp.int32), jnp.ones((NCAND,), jnp.int32)])
    ks, ts = lax.sort([kall, tag], num_keys=1)
    prev = jnp.concatenate([jnp.full((1,), -1, jnp.int32), ks[:-1]])
    free = jnp.logical_and(ts == 1, ks != prev)
    safe = lax.sort([jnp.where(free, ks, jnp.int32(1 << 30))], num_keys=1)[0]
    run_rows = jnp.where(iota < nruns, run_rows, safe[:NIDX])
    qid = rid // SLAB
    q = jnp.arange(NSLABS_TOT, dtype=jnp.int32)[:, None]
    ps = jnp.sum((qid[None, :] < q).astype(jnp.int32), axis=1)
    pe = jnp.sum((qid[None, :] <= q).astype(jnp.int32), axis=1)
    recs = jnp.stack([ps, pe], axis=-1)
    recs = jnp.pad(recs, ((0, 0), (0, PREC - 2))).reshape(-1)

    out1 = _tc_copy(A)
    ref = jax.new_ref(out1)
    _sc_scatter(order, rid, run_rows, recs, B, ref)
    return jax.freeze(ref)


def kernel(index, A, B):
    return _scatter_add(index.astype(jnp.int32), A, B)


# speedup vs baseline: 13.3986x; 13.3986x over previous
import jax
import jax.numpy as jnp
from jax import lax
from jax.experimental import pallas as pl
from jax.experimental.pallas import tpu as pltpu
from jax.experimental.pallas import tpu_sc as plsc

ROWS = 1_000_000
D = 64
NIDX = 16384

NC = 2
NS = 16
NW = NC * NS
RUNS_PER_TILE = NIDX // NW
SLAB = 128
NSLAB = RUNS_PER_TILE // SLAB
NSLABS_TOT = NIDX // SLAB
PREC = 16
NCAND = 2 * NIDX + 16

COPY_CHUNK = 125_000
NCHUNK = ROWS // COPY_CHUNK


def _lane(vec, j):
    return lax.squeeze(lax.slice(vec, [j], [j + 1]), [0])


def _sc_body(order_hbm, rid_hbm, rr_hbm, rec_hbm, b_hbm, out_hbm,
             odbuf, ridbuf, recbuf, rbuf, tbuf, bbuf, gbuf, semB):
    wid = lax.axis_index("s") * NC + lax.axis_index("c")

    pltpu.sync_copy(order_hbm, odbuf)
    pltpu.sync_copy(rid_hbm, ridbuf)
    pltpu.sync_copy(rec_hbm.at[pl.ds(wid * NSLAB * PREC, NSLAB * PREC)],
                    recbuf)
    zeros16 = jnp.zeros((16,), jnp.float32)

    def slab_body(sl, carry):
        rec = recbuf[pl.ds(sl * PREC, 16)]
        ps = _lane(rec, 0)
        pe = _lane(rec, 1)
        rid0 = wid * RUNS_PER_TILE + sl * SLAB

        def zero_body(r, zc):
            for cg in range(4):
                tbuf[r, pl.ds(cg * 16, 16)] = zeros16
            return zc
        lax.fori_loop(0, SLAB, zero_body, 0)

        pltpu.sync_copy(rr_hbm.at[pl.ds(rid0, SLAB)], rbuf)

        def batch_body(b, bc):
            bb = b * SLAB
            pltpu.async_copy(
                b_hbm.at[odbuf.at[pl.ds(bb, SLAB)]], bbuf, semB).wait()
            for sub in range(SLAB // 16):
                rv = ridbuf[pl.ds(bb + sub * 16, 16)]
                for j in range(16):
                    pos = bb + sub * 16 + j
                    cond = jnp.logical_and(pos >= ps, pos < pe)

                    @pl.when(cond)
                    def _(sub=sub, j=j, rv=rv):
                        lr = _lane(rv, j) - rid0
                        for cg in range(4):
                            x = bbuf[sub * 16 + j, pl.ds(cg * 16, 16)]
                            plsc.addupdate(
                                tbuf.at[lr, pl.ds(cg * 16, 16)], x)
            return bc

        lax.fori_loop(ps // SLAB, (pe + SLAB - 1) // SLAB, batch_body, 0)

        pltpu.async_copy(out_hbm.at[rbuf], gbuf, semB).wait()

        def add_body(r, ac):
            for cg in range(4):
                x = gbuf[r, pl.ds(cg * 16, 16)]
                plsc.addupdate(tbuf.at[r, pl.ds(cg * 16, 16)], x)
            return ac
        lax.fori_loop(0, SLAB, add_body, 0)

        pltpu.async_copy(tbuf, out_hbm.at[rbuf], semB).wait()
        return carry

    lax.fori_loop(0, NSLAB, slab_body, 0)


def _sc_scatter(order, rid, run_rows, recs, B, out_ref):
    mesh = plsc.VectorSubcoreMesh(
        core_axis_name="c", subcore_axis_name="s",
        num_cores=NC, num_subcores=NS)
    f = pl.kernel(
        _sc_body,
        out_type=(),
        mesh=mesh,
        compiler_params=pltpu.CompilerParams(use_tc_tiling_on_sc=False),
        scratch_types=[
            pltpu.VMEM((NIDX,), jnp.int32),
            pltpu.VMEM((NIDX,), jnp.int32),
            pltpu.VMEM((NSLAB * PREC,), jnp.int32),
            pltpu.VMEM((SLAB,), jnp.int32),
            pltpu.VMEM((SLAB, D), jnp.float32),
            pltpu.VMEM((SLAB, D), jnp.float32),
            pltpu.VMEM((SLAB, D), jnp.float32),
            pltpu.SemaphoreType.DMA,
        ],
    )
    f(order, rid, run_rows, recs, B, out_ref)


@jax.jit
def _scatter_add(index, A, B):
    iota = jnp.arange(NIDX, dtype=jnp.int32)
    sidx, order = lax.sort([index, iota], num_keys=1)
    is_start = jnp.concatenate(
        [jnp.ones((1,), jnp.bool_), sidx[1:] != sidx[:-1]])
    rid = jnp.cumsum(is_start.astype(jnp.int32)) - 1
    nruns = rid[NIDX - 1] + 1
    keys = jnp.where(is_start, rid, NIDX)
    run_rows = lax.sort([keys, sidx], num_keys=1)[1]
    cand = jnp.arange(NCAND, dtype=jnp.int32)
    kall = jnp.concatenate([sidx, cand])
    tag = jnp.concatenate(
        [jnp.zeros((NIDX,), jnp.int32), jnp.ones((NCAND,), jnp.int32)])
    ks, ts = lax.sort([kall, tag], num_keys=1)
    prev = jnp.concatenate([jnp.full((1,), -1, jnp.int32), ks[:-1]])
    free = jnp.logical_and(ts == 1, ks != prev)
    safe = lax.sort([jnp.where(free, ks, jnp.int32(1 << 30))], num_keys=1)[0]
    run_rows = jnp.where(iota < nruns, run_rows, safe[:NIDX])
    qid = rid // SLAB
    q = jnp.arange(NSLABS_TOT, dtype=jnp.int32)[:, None]
    ps = jnp.sum((qid[None, :] < q).astype(jnp.int32), axis=1)
    pe = jnp.sum((qid[None, :] <= q).astype(jnp.int32), axis=1)
    recs = jnp.stack([ps, pe], axis=-1)
    recs = jnp.pad(recs, ((0, 0), (0, PREC - 2))).reshape(-1)

    ref = jax.new_ref(A)
    _sc_scatter(order, rid, run_rows, recs, B, ref)
    return jax.freeze(ref)


def kernel(index, A, B):
    return _scatter_add(index.astype(jnp.int32), A, B)


# speedup vs baseline: 13.5428x; 1.0108x over previous
import jax
import jax.numpy as jnp
from jax import lax
from jax.experimental import pallas as pl
from jax.experimental.pallas import tpu as pltpu
from jax.experimental.pallas import tpu_sc as plsc

ROWS = 1_000_000
D = 64
NIDX = 16384
G8 = 8

NC = 2
NS = 16
NW = NC * NS
RUNS_PER_TILE = NIDX // NW
SLAB = 64
NSLAB = RUNS_PER_TILE // SLAB
NSLABS_TOT = NIDX // SLAB
PB = 128
PREC = 16
NCAND = 2 * NIDX + 16


def _lane(vec, j):
    return lax.squeeze(lax.slice(vec, [j], [j + 1]), [0])


def _sc_body(sidx_hbm, rid_hbm, order_hbm, rr_hbm, rec_hbm, b_hbm, a_hbm,
             out_hbm, odbuf, recbuf, rbuf, gacc, bbuf, sxb, ridb,
             semG, semB):
    wid = lax.axis_index("s") * NC + lax.axis_index("c")

    pltpu.sync_copy(order_hbm, odbuf)
    pltpu.sync_copy(rec_hbm.at[pl.ds(wid * NSLAB * PREC, NSLAB * PREC)],
                    recbuf)

    def slab_body(sl, carry):
        rec = recbuf[pl.ds(sl * PREC, 16)]
        ps = _lane(rec, 0)
        pe = _lane(rec, 1)
        rid0 = wid * RUNS_PER_TILE + sl * SLAB
        pltpu.sync_copy(rr_hbm.at[pl.ds(rid0, SLAB)], rbuf)

        for b16 in range(SLAB // 16):
            gvec = rbuf[pl.ds(b16 * 16, 16)]
            for l in range(16):
                g = _lane(gvec, l)
                k = b16 * 16 + l
                pltpu.async_copy(
                    a_hbm.at[pl.ds(g * G8, G8)],
                    gacc.at[pl.ds(k * G8, G8)], semG)
        for k in range(SLAB):
            pltpu.make_async_copy(
                a_hbm.at[pl.ds(0, G8)], gacc.at[pl.ds(0, G8)], semG).wait()

        def batch_body(b, bc):
            bb = b * PB
            pltpu.sync_copy(sidx_hbm.at[pl.ds(bb, PB)], sxb)
            pltpu.sync_copy(rid_hbm.at[pl.ds(bb, PB)], ridb)
            pltpu.async_copy(
                b_hbm.at[odbuf.at[pl.ds(bb, PB)]], bbuf, semB).wait()
            for sub in range(PB // 16):
                rv = ridb[pl.ds(sub * 16, 16)]
                sv = sxb[pl.ds(sub * 16, 16)]
                for j in range(16):
                    pos = bb + sub * 16 + j
                    cond = jnp.logical_and(pos >= ps, pos < pe)

                    @pl.when(cond)
                    def _(sub=sub, j=j, rv=rv, sv=sv):
                        lr = _lane(rv, j) - rid0
                        r8 = _lane(sv, j) & (G8 - 1)
                        lg = lr * G8 + r8
                        for cg in range(4):
                            x = bbuf[sub * 16 + j, pl.ds(cg * 16, 16)]
                            plsc.addupdate(
                                gacc.at[lg, pl.ds(cg * 16, 16)], x)
            return bc

        lax.fori_loop(ps // PB, (pe + PB - 1) // PB, batch_body, 0)

        for b16 in range(SLAB // 16):
            gvec = rbuf[pl.ds(b16 * 16, 16)]
            for l in range(16):
                g = _lane(gvec, l)
                k = b16 * 16 + l
                pltpu.async_copy(
                    gacc.at[pl.ds(k * G8, G8)],
                    out_hbm.at[pl.ds(g * G8, G8)], semG)
        for k in range(SLAB):
            pltpu.make_async_copy(
                gacc.at[pl.ds(0, G8)], out_hbm.at[pl.ds(0, G8)],
                semG).wait()
        return carry

    lax.fori_loop(0, NSLAB, slab_body, 0)


def _sc_scatter(sidx, rid, order, run_gids, recs, B_pad, A, out_ref):
    mesh = plsc.VectorSubcoreMesh(
        core_axis_name="c", subcore_axis_name="s",
        num_cores=NC, num_subcores=NS)
    f = pl.kernel(
        _sc_body,
        out_type=(),
        mesh=mesh,
        scratch_types=[
            pltpu.VMEM((NIDX,), jnp.int32),
            pltpu.VMEM((NSLAB * PREC,), jnp.int32),
            pltpu.VMEM((SLAB,), jnp.int32),
            pltpu.VMEM((SLAB * G8, D), jnp.float32),
            pltpu.VMEM((PB, 2 * D), jnp.float32),
            pltpu.VMEM((PB,), jnp.int32),
            pltpu.VMEM((PB,), jnp.int32),
            pltpu.SemaphoreType.DMA,
            pltpu.SemaphoreType.DMA,
        ],
    )
    f(sidx, rid, order, run_gids, recs, B_pad, A, out_ref)


@jax.jit
def _scatter_add(index, A, B):
    iota = jnp.arange(NIDX, dtype=jnp.int32)
    sidx, order = lax.sort([index, iota], num_keys=1)
    sgid = sidx // G8
    is_start = jnp.concatenate(
        [jnp.ones((1,), jnp.bool_), sgid[1:] != sgid[:-1]])
    rid = jnp.cumsum(is_start.astype(jnp.int32)) - 1
    nruns = rid[NIDX - 1] + 1
    keys = jnp.where(is_start, rid, NIDX)
    run_gids = lax.sort([keys, sgid], num_keys=1)[1]
    cand = jnp.arange(NCAND, dtype=jnp.int32)
    kall = jnp.concatenate([sgid, cand])
    tag = jnp.concatenate(
        [jnp.zeros((NIDX,), jnp.int32), jnp.ones((NCAND,), jnp.int32)])
    ks, ts = lax.sort([kall, tag], num_keys=1)
    prev = jnp.concatenate([jnp.full((1,), -1, jnp.int32), ks[:-1]])
    free = jnp.logical_and(ts == 1, ks != prev)
    safe = lax.sort([jnp.where(free, ks, jnp.int32(1 << 30))], num_keys=1)[0]
    run_gids = jnp.where(iota < nruns, run_gids, safe[:NIDX])
    qid = rid // SLAB
    q = jnp.arange(NSLABS_TOT, dtype=jnp.int32)[:, None]
    ps = jnp.sum((qid[None, :] < q).astype(jnp.int32), axis=1)
    pe = jnp.sum((qid[None, :] <= q).astype(jnp.int32), axis=1)
    recs = jnp.stack([ps, pe], axis=-1)
    recs = jnp.pad(recs, ((0, 0), (0, PREC - 2))).reshape(-1)

    B_pad = jnp.pad(B, ((0, 0), (0, D)))
    ref = jax.new_ref(A)
    _sc_scatter(sidx, rid, order, run_gids, recs, B_pad, A, ref)
    return jax.freeze(ref)


def kernel(index, A, B):
    return _scatter_add(index.astype(jnp.int32), A, B)
